# baseline (speedup 1.0000x reference)
import jax
import jax.numpy as jnp
from jax.experimental import pallas as pl


def _identity_body(x_ref, o_ref):
    o_ref[...] = x_ref[...]


def _pallas_identity(x):
    n = x.shape[0]
    x2 = x.reshape(n // 128, 128)
    out = pl.pallas_call(
        _identity_body,
        out_shape=jax.ShapeDtypeStruct(x2.shape, x2.dtype),
        grid=(50,),
        in_specs=[pl.BlockSpec((n // 128 // 50, 128), lambda i: (i, 0))],
        out_specs=pl.BlockSpec((n // 128 // 50, 128), lambda i: (i, 0)),
    )(x2)
    return out.reshape(x.shape)


def kernel(pr, vr):
    G = 64
    sg = jax.lax.stop_gradient
    vr2 = jnp.squeeze(sg(vr), -1)
    seg = vr2.reshape(-1)
    flat = pr.reshape(-1)
    sums = jax.ops.segment_sum(flat, seg, num_segments=G)
    cnts = jax.ops.segment_sum(jnp.ones_like(flat), seg, num_segments=G)
    vmeans = sums / cnts
    vmins = jax.ops.segment_min(flat, seg, num_segments=G)
    vmaxs = jax.ops.segment_max(flat, seg, num_segments=G)
    no_scale_idx = jnp.equal(vmins, vmaxs)
    vmins_g = jnp.where(no_scale_idx, jnp.float32(0.0), vmins)[vr2]
    vmaxs_g = jnp.where(no_scale_idx, jnp.float32(1.0), vmaxs)[vr2]
    nan_idx = jnp.isnan(vmeans)
    min_valid = jnp.min(jnp.where(nan_idx, jnp.inf, vmeans))
    vmeans = jnp.where(nan_idx, min_valid / 2.0, vmeans)
    sort_ids = jnp.argsort(vmeans)
    orig_ids = jnp.argsort(sort_ids)
    vmeans_sorted = vmeans[sort_ids]
    conv_data = jnp.concatenate(
        [vmeans_sorted[:1], vmeans_sorted, vmeans_sorted[-1:] * 2.0], axis=0)
    s = conv_data[:-1] + conv_data[1:]
    f0 = (s / 1.99)[:-1]
    f1 = (s / 2.01)[1:]
    f0 = jnp.where(no_scale_idx, jnp.float32(0.0), f0[orig_ids])[vr2]
    f1 = jnp.where(no_scale_idx, jnp.float32(1.0), f1[orig_ids])[vr2]
    tmp = (sg(pr) - sg(vmins_g)) / sg(vmaxs_g - vmins_g) * sg(f1 - f0) + sg(f0)
    scale = pr / tmp
    scale = jnp.where(jnp.isnan(scale) | jnp.isinf(scale), jnp.float32(0.0), scale)
    res = pr * sg(scale)
    return _pallas_identity(res)


# speedup vs baseline: 4.1336x; 4.1336x over previous
import functools

import jax
import jax.numpy as jnp
from jax import lax
from jax.experimental import pallas as pl
from jax.experimental.pallas import tpu as pltpu
from jax.experimental.pallas import tpu_sc as plsc

G = 64
N = 6400000
NW = 32
PER_W = N // NW
CHUNK = 10000
NCHUNK = PER_W // CHUNK
VECS = CHUNK // 16

_MESH = plsc.VectorSubcoreMesh(core_axis_name="c", subcore_axis_name="s")


def _wid():
    return lax.axis_index("s") * 2 + lax.axis_index("c")


@functools.partial(
    pl.kernel,
    mesh=_MESH,
    compiler_params=pltpu.CompilerParams(needs_layout_passes=False),
    out_type=[
        jax.ShapeDtypeStruct((NW, 1, G * 16), jnp.float32),
        jax.ShapeDtypeStruct((NW, 1, G * 16), jnp.float32),
        jax.ShapeDtypeStruct((NW, 1, G * 16), jnp.float32),
    ],
    scratch_types=[
        pltpu.VMEM((CHUNK,), jnp.float32),
        pltpu.VMEM((CHUNK,), jnp.int32),
        pltpu.VMEM((G * 16,), jnp.float32),
        pltpu.VMEM((G * 16,), jnp.float32),
        pltpu.VMEM((G * 16,), jnp.float32),
    ],
)
def _sc_pass1(pr_hbm, vr_hbm, cnt_hbm, min_hbm, max_hbm,
              prbuf, vrbuf, cnt_tab, min_tab, max_tab):
    wid = _wid()
    base = wid * PER_W
    inf = jnp.full((16,), jnp.inf, jnp.float32)
    zero = jnp.zeros((16,), jnp.float32)
    for g in range(G):
        cnt_tab[pl.ds(g * 16, 16)] = zero
        min_tab[pl.ds(g * 16, 16)] = inf
        max_tab[pl.ds(g * 16, 16)] = -inf
    lanes = lax.iota(jnp.int32, 16)
    ones = jnp.ones((16,), jnp.float32)

    def chunk_body(c, _):
        off = base + c * CHUNK
        pltpu.sync_copy(pr_hbm.at[pl.ds(off, CHUNK)], prbuf)
        pltpu.sync_copy(vr_hbm.at[pl.ds(off, CHUNK)], vrbuf)

        def vec_body(i, _):
            x = prbuf[pl.ds(i * 16, 16)]
            idx = vrbuf[pl.ds(i * 16, 16)] * 16 + lanes
            plsc.addupdate_scatter(cnt_tab, [idx], ones)
            cur_mn = plsc.load_gather(min_tab, [idx])
            plsc.store_scatter(min_tab, [idx], jnp.minimum(cur_mn, x))
            cur_mx = plsc.load_gather(max_tab, [idx])
            plsc.store_scatter(max_tab, [idx], jnp.maximum(cur_mx, x))
            return 0

        lax.fori_loop(0, VECS, vec_body, 0)
        return 0

    lax.fori_loop(0, NCHUNK, chunk_body, 0)
    pltpu.sync_copy(cnt_tab, cnt_hbm.at[wid, 0])
    pltpu.sync_copy(min_tab, min_hbm.at[wid, 0])
    pltpu.sync_copy(max_tab, max_hbm.at[wid, 0])


@functools.partial(
    pl.kernel,
    mesh=_MESH,
    compiler_params=pltpu.CompilerParams(needs_layout_passes=False),
    out_type=jax.ShapeDtypeStruct((N,), jnp.float32),
    scratch_types=[
        pltpu.VMEM((CHUNK,), jnp.float32),
        pltpu.VMEM((CHUNK,), jnp.int32),
        pltpu.VMEM((CHUNK,), jnp.float32),
        pltpu.VMEM((G,), jnp.float32),
        pltpu.VMEM((G,), jnp.float32),
        pltpu.VMEM((G,), jnp.float32),
        pltpu.VMEM((G,), jnp.float32),
    ],
)
def _sc_map(pr_hbm, vr_hbm, tmin_hbm, tmax_hbm, tf0_hbm, tf1_hbm, out_hbm,
            prbuf, vrbuf, obuf, tmin, tmax, tf0, tf1):
    wid = _wid()
    base = wid * PER_W
    pltpu.sync_copy(tmin_hbm, tmin)
    pltpu.sync_copy(tmax_hbm, tmax)
    pltpu.sync_copy(tf0_hbm, tf0)
    pltpu.sync_copy(tf1_hbm, tf1)

    def chunk_body(c, _):
        off = base + c * CHUNK
        pltpu.sync_copy(pr_hbm.at[pl.ds(off, CHUNK)], prbuf)
        pltpu.sync_copy(vr_hbm.at[pl.ds(off, CHUNK)], vrbuf)

        def vec_body(i, _):
            sl = pl.ds(i * 16, 16)
            x = prbuf[sl]
            idx = vrbuf[sl]
            vmn = plsc.load_gather(tmin, [idx])
            vmx = plsc.load_gather(tmax, [idx])
            f0 = plsc.load_gather(tf0, [idx])
            f1 = plsc.load_gather(tf1, [idx])
            tmp = (x - vmn) / (vmx - vmn) * (f1 - f0) + f0
            sc = x / tmp
            bad = (sc != sc) | (jnp.abs(sc) == jnp.inf)
            sc = jnp.where(bad, jnp.float32(0.0), sc)
            obuf[sl] = x * sc
            return 0

        lax.fori_loop(0, VECS, vec_body, 0)
        pltpu.sync_copy(obuf, out_hbm.at[pl.ds(off, CHUNK)])
        return 0

    lax.fori_loop(0, NCHUNK, chunk_body, 0)


def kernel(pr, vr):
    flat = pr.reshape(-1)
    seg = vr.reshape(-1)
    sums = jax.ops.segment_sum(flat, seg, num_segments=G)
    p_cnt, p_min, p_max = _sc_pass1(flat, seg)
    cnts = jnp.sum(p_cnt.reshape(NW, G, 16), axis=(0, 2))
    vmins = jnp.min(p_min.reshape(NW, G, 16), axis=(0, 2))
    vmaxs = jnp.max(p_max.reshape(NW, G, 16), axis=(0, 2))

    vmeans = sums / cnts
    no_scale = jnp.equal(vmins, vmaxs)
    nan_idx = jnp.isnan(vmeans)
    min_valid = jnp.min(jnp.where(nan_idx, jnp.inf, vmeans))
    vmeans = jnp.where(nan_idx, min_valid / 2.0, vmeans)
    sort_ids = jnp.argsort(vmeans)
    orig_ids = jnp.argsort(sort_ids)
    vs = vmeans[sort_ids]
    conv = jnp.concatenate([vs[:1], vs, vs[-1:] * 2.0], axis=0)
    s = conv[:-1] + conv[1:]
    f0 = (s / 1.99)[:-1]
    f1 = (s / 2.01)[1:]
    zf = jnp.float32(0.0)
    of = jnp.float32(1.0)
    t_vmin = jnp.where(no_scale, zf, vmins)
    t_vmax = jnp.where(no_scale, of, vmaxs)
    t_f0 = jnp.where(no_scale, zf, f0[orig_ids])
    t_f1 = jnp.where(no_scale, of, f1[orig_ids])

    res = _sc_map(flat, seg, t_vmin, t_vmax, t_f0, t_f1)
    return res.reshape(pr.shape)


# speedup vs baseline: 4.1425x; 1.0022x over previous
import functools

import jax
import jax.numpy as jnp
from jax import lax
from jax.experimental import pallas as pl
from jax.experimental.pallas import tpu as pltpu
from jax.experimental.pallas import tpu_sc as plsc

G = 64
N = 6400000
NW = 32
PER_W = N // NW
CHUNK = 40000
NCHUNK = PER_W // CHUNK
VECS = CHUNK // 16

_MESH = plsc.VectorSubcoreMesh(core_axis_name="c", subcore_axis_name="s")


def _wid():
    return lax.axis_index("s") * 2 + lax.axis_index("c")


@functools.partial(
    pl.kernel,
    mesh=_MESH,
    compiler_params=pltpu.CompilerParams(needs_layout_passes=False),
    out_type=[
        jax.ShapeDtypeStruct((NW, 1, G * 16), jnp.float32),
        jax.ShapeDtypeStruct((NW, 1, G * 16), jnp.float32),
        jax.ShapeDtypeStruct((NW, 1, G * 16), jnp.float32),
    ],
    scratch_types=[
        pltpu.VMEM((CHUNK,), jnp.float32),
        pltpu.VMEM((CHUNK,), jnp.int32),
        pltpu.VMEM((G * 16,), jnp.float32),
        pltpu.VMEM((G * 16,), jnp.float32),
        pltpu.VMEM((G * 16,), jnp.float32),
    ],
)
def _sc_pass1(pr_hbm, vr_hbm, cnt_hbm, min_hbm, max_hbm,
              prbuf, vrbuf, cnt_tab, min_tab, max_tab):
    wid = _wid()
    base = wid * PER_W
    inf = jnp.full((16,), jnp.inf, jnp.float32)
    zero = jnp.zeros((16,), jnp.float32)
    for g in range(G):
        cnt_tab[pl.ds(g * 16, 16)] = zero
        min_tab[pl.ds(g * 16, 16)] = inf
        max_tab[pl.ds(g * 16, 16)] = -inf
    lanes = lax.iota(jnp.int32, 16)
    ones = jnp.ones((16,), jnp.float32)

    def chunk_body(c, _):
        off = base + c * CHUNK
        pltpu.sync_copy(pr_hbm.at[pl.ds(off, CHUNK)], prbuf)
        pltpu.sync_copy(vr_hbm.at[pl.ds(off, CHUNK)], vrbuf)

        def vec_body(i, _):
            x = prbuf[pl.ds(i * 16, 16)]
            idx = vrbuf[pl.ds(i * 16, 16)] * 16 + lanes
            plsc.addupdate_scatter(cnt_tab, [idx], ones)
            cur_mn = plsc.load_gather(min_tab, [idx])
            plsc.store_scatter(min_tab, [idx], jnp.minimum(cur_mn, x))
            cur_mx = plsc.load_gather(max_tab, [idx])
            plsc.store_scatter(max_tab, [idx], jnp.maximum(cur_mx, x))
            return 0

        lax.fori_loop(0, VECS, vec_body, 0)
        return 0

    lax.fori_loop(0, NCHUNK, chunk_body, 0)
    pltpu.sync_copy(cnt_tab, cnt_hbm.at[wid, 0])
    pltpu.sync_copy(min_tab, min_hbm.at[wid, 0])
    pltpu.sync_copy(max_tab, max_hbm.at[wid, 0])


@functools.partial(
    pl.kernel,
    mesh=_MESH,
    compiler_params=pltpu.CompilerParams(needs_layout_passes=False),
    out_type=jax.ShapeDtypeStruct((N,), jnp.float32),
    scratch_types=[
        pltpu.VMEM((CHUNK,), jnp.float32),
        pltpu.VMEM((CHUNK,), jnp.int32),
        pltpu.VMEM((CHUNK,), jnp.float32),
        pltpu.VMEM((G,), jnp.float32),
        pltpu.VMEM((G,), jnp.float32),
        pltpu.VMEM((G,), jnp.float32),
        pltpu.VMEM((G,), jnp.float32),
    ],
)
def _sc_map(pr_hbm, vr_hbm, tmin_hbm, tmax_hbm, tf0_hbm, tf1_hbm, out_hbm,
            prbuf, vrbuf, obuf, tmin, tmax, tf0, tf1):
    wid = _wid()
    base = wid * PER_W
    pltpu.sync_copy(tmin_hbm, tmin)
    pltpu.sync_copy(tmax_hbm, tmax)
    pltpu.sync_copy(tf0_hbm, tf0)
    pltpu.sync_copy(tf1_hbm, tf1)

    def chunk_body(c, _):
        off = base + c * CHUNK
        pltpu.sync_copy(pr_hbm.at[pl.ds(off, CHUNK)], prbuf)
        pltpu.sync_copy(vr_hbm.at[pl.ds(off, CHUNK)], vrbuf)

        def vec_body(i, _):
            sl = pl.ds(i * 16, 16)
            x = prbuf[sl]
            idx = vrbuf[sl]
            vmn = plsc.load_gather(tmin, [idx])
            vmx = plsc.load_gather(tmax, [idx])
            f0 = plsc.load_gather(tf0, [idx])
            f1 = plsc.load_gather(tf1, [idx])
            tmp = (x - vmn) / (vmx - vmn) * (f1 - f0) + f0
            sc = x / tmp
            bad = (sc != sc) | (jnp.abs(sc) == jnp.inf)
            sc = jnp.where(bad, jnp.float32(0.0), sc)
            obuf[sl] = x * sc
            return 0

        lax.fori_loop(0, VECS, vec_body, 0)
        pltpu.sync_copy(obuf, out_hbm.at[pl.ds(off, CHUNK)])
        return 0

    lax.fori_loop(0, NCHUNK, chunk_body, 0)


def kernel(pr, vr):
    flat = pr.reshape(-1)
    seg = vr.reshape(-1)
    sums = jax.ops.segment_sum(flat, seg, num_segments=G)
    p_cnt, p_min, p_max = _sc_pass1(flat, seg)
    cnts = jnp.sum(p_cnt.reshape(NW, G, 16), axis=(0, 2))
    vmins = jnp.min(p_min.reshape(NW, G, 16), axis=(0, 2))
    vmaxs = jnp.max(p_max.reshape(NW, G, 16), axis=(0, 2))

    vmeans = sums / cnts
    no_scale = jnp.equal(vmins, vmaxs)
    nan_idx = jnp.isnan(vmeans)
    min_valid = jnp.min(jnp.where(nan_idx, jnp.inf, vmeans))
    vmeans = jnp.where(nan_idx, min_valid / 2.0, vmeans)
    sort_ids = jnp.argsort(vmeans)
    orig_ids = jnp.argsort(sort_ids)
    vs = vmeans[sort_ids]
    conv = jnp.concatenate([vs[:1], vs, vs[-1:] * 2.0], axis=0)
    s = conv[:-1] + conv[1:]
    f0 = (s / 1.99)[:-1]
    f1 = (s / 2.01)[1:]
    zf = jnp.float32(0.0)
    of = jnp.float32(1.0)
    t_vmin = jnp.where(no_scale, zf, vmins)
    t_vmax = jnp.where(no_scale, of, vmaxs)
    t_f0 = jnp.where(no_scale, zf, f0[orig_ids])
    t_f1 = jnp.where(no_scale, of, f1[orig_ids])

    res = _sc_map(flat, seg, t_vmin, t_vmax, t_f0, t_f1)
    return res.reshape(pr.shape)


# speedup vs baseline: 4.2914x; 1.0359x over previous
import functools

import jax
import jax.numpy as jnp
from jax import lax
from jax.experimental import pallas as pl
from jax.experimental.pallas import tpu as pltpu
from jax.experimental.pallas import tpu_sc as plsc

G = 64
N = 6400000
NW = 32
PER_W = N // NW
CHUNK = 40000
NCHUNK = PER_W // CHUNK
VECS = CHUNK // 16

_MESH = plsc.VectorSubcoreMesh(core_axis_name="c", subcore_axis_name="s")


def _wid():
    return lax.axis_index("s") * 2 + lax.axis_index("c")


@functools.partial(
    pl.kernel,
    mesh=_MESH,
    compiler_params=pltpu.CompilerParams(needs_layout_passes=False),
    out_type=[
        jax.ShapeDtypeStruct((NW, 1, G * 16), jnp.float32),
        jax.ShapeDtypeStruct((NW, 1, G * 16), jnp.float32),
        jax.ShapeDtypeStruct((NW, 1, G * 16), jnp.float32),
    ],
    scratch_types=[
        pltpu.VMEM((CHUNK,), jnp.float32),
        pltpu.VMEM((CHUNK,), jnp.int32),
        pltpu.VMEM((G * 16,), jnp.float32),
        pltpu.VMEM((G * 16,), jnp.float32),
        pltpu.VMEM((G * 16,), jnp.float32),
    ],
)
def _sc_pass1(pr_hbm, vr_hbm, cnt_hbm, min_hbm, max_hbm,
              prbuf, vrbuf, cnt_tab, min_tab, max_tab):
    wid = _wid()
    base = wid * PER_W
    inf = jnp.full((16,), jnp.inf, jnp.float32)
    zero = jnp.zeros((16,), jnp.float32)
    for g in range(G):
        cnt_tab[pl.ds(g * 16, 16)] = zero
        min_tab[pl.ds(g * 16, 16)] = inf
        max_tab[pl.ds(g * 16, 16)] = -inf
    lanes = lax.iota(jnp.int32, 16)
    ones = jnp.ones((16,), jnp.float32)

    def chunk_body(c, _):
        off = base + c * CHUNK
        pltpu.sync_copy(pr_hbm.at[pl.ds(off, CHUNK)], prbuf)
        pltpu.sync_copy(vr_hbm.at[pl.ds(off, CHUNK)], vrbuf)

        def vec_body(i, _):
            x = prbuf[pl.ds(i * 16, 16)]
            idx = vrbuf[pl.ds(i * 16, 16)] * 16 + lanes
            plsc.addupdate_scatter(cnt_tab, [idx], ones)
            cur_mn = plsc.load_gather(min_tab, [idx])
            plsc.store_scatter(min_tab, [idx], jnp.minimum(cur_mn, x))
            cur_mx = plsc.load_gather(max_tab, [idx])
            plsc.store_scatter(max_tab, [idx], jnp.maximum(cur_mx, x))
            return 0

        lax.fori_loop(0, VECS, vec_body, 0)
        return 0

    lax.fori_loop(0, NCHUNK, chunk_body, 0)
    pltpu.sync_copy(cnt_tab, cnt_hbm.at[wid, 0])
    pltpu.sync_copy(min_tab, min_hbm.at[wid, 0])
    pltpu.sync_copy(max_tab, max_hbm.at[wid, 0])


@functools.partial(
    pl.kernel,
    mesh=_MESH,
    compiler_params=pltpu.CompilerParams(needs_layout_passes=False),
    out_type=jax.ShapeDtypeStruct((N,), jnp.float32),
    scratch_types=[
        pltpu.VMEM((CHUNK,), jnp.float32),
        pltpu.VMEM((CHUNK,), jnp.int32),
        pltpu.VMEM((CHUNK,), jnp.float32),
        pltpu.VMEM((G,), jnp.float32),
        pltpu.VMEM((G,), jnp.float32),
        pltpu.VMEM((G,), jnp.float32),
        pltpu.VMEM((G,), jnp.float32),
    ],
)
def _sc_map(pr_hbm, vr_hbm, tmin_hbm, tmax_hbm, tf0_hbm, tf1_hbm, out_hbm,
            prbuf, vrbuf, obuf, tmin, tmax, tf0, tf1):
    wid = _wid()
    base = wid * PER_W
    pltpu.sync_copy(tmin_hbm, tmin)
    pltpu.sync_copy(tmax_hbm, tmax)
    pltpu.sync_copy(tf0_hbm, tf0)
    pltpu.sync_copy(tf1_hbm, tf1)

    def chunk_body(c, _):
        off = base + c * CHUNK
        pltpu.sync_copy(pr_hbm.at[pl.ds(off, CHUNK)], prbuf)
        pltpu.sync_copy(vr_hbm.at[pl.ds(off, CHUNK)], vrbuf)

        @plsc.parallel_loop(0, CHUNK, step=16, unroll=4)
        def vec_body(i):
            sl = pl.ds(i, 16)
            x = prbuf[sl]
            idx = vrbuf[sl]
            vmn = plsc.load_gather(tmin, [idx])
            vmx = plsc.load_gather(tmax, [idx])
            f0 = plsc.load_gather(tf0, [idx])
            f1 = plsc.load_gather(tf1, [idx])
            tmp = (x - vmn) / (vmx - vmn) * (f1 - f0) + f0
            sc = x / tmp
            bad = (sc != sc) | (jnp.abs(sc) == jnp.inf)
            sc = jnp.where(bad, jnp.float32(0.0), sc)
            obuf[sl] = x * sc
        pltpu.sync_copy(obuf, out_hbm.at[pl.ds(off, CHUNK)])
        return 0

    lax.fori_loop(0, NCHUNK, chunk_body, 0)


def kernel(pr, vr):
    flat = pr.reshape(-1)
    seg = vr.reshape(-1)
    sums = jax.ops.segment_sum(flat, seg, num_segments=G)
    p_cnt, p_min, p_max = _sc_pass1(flat, seg)
    cnts = jnp.sum(p_cnt.reshape(NW, G, 16), axis=(0, 2))
    vmins = jnp.min(p_min.reshape(NW, G, 16), axis=(0, 2))
    vmaxs = jnp.max(p_max.reshape(NW, G, 16), axis=(0, 2))

    vmeans = sums / cnts
    no_scale = jnp.equal(vmins, vmaxs)
    nan_idx = jnp.isnan(vmeans)
    min_valid = jnp.min(jnp.where(nan_idx, jnp.inf, vmeans))
    vmeans = jnp.where(nan_idx, min_valid / 2.0, vmeans)
    sort_ids = jnp.argsort(vmeans)
    orig_ids = jnp.argsort(sort_ids)
    vs = vmeans[sort_ids]
    conv = jnp.concatenate([vs[:1], vs, vs[-1:] * 2.0], axis=0)
    s = conv[:-1] + conv[1:]
    f0 = (s / 1.99)[:-1]
    f1 = (s / 2.01)[1:]
    zf = jnp.float32(0.0)
    of = jnp.float32(1.0)
    t_vmin = jnp.where(no_scale, zf, vmins)
    t_vmax = jnp.where(no_scale, of, vmaxs)
    t_f0 = jnp.where(no_scale, zf, f0[orig_ids])
    t_f1 = jnp.where(no_scale, of, f1[orig_ids])

    res = _sc_map(flat, seg, t_vmin, t_vmax, t_f0, t_f1)
    return res.reshape(pr.shape)
